# baseline (speedup 1.0000x reference)
import functools

import jax
import jax.numpy as jnp
from jax import lax
from jax.experimental import pallas as pl
from jax.experimental.pallas import tpu as pltpu
from jax.experimental.pallas import tpu_sc as plsc

_B, _L = 4096, 50
_N = _B * _L
_D = 64
_V = 100000
_VP = 100096
_NW = 32
_T = _N // _NW
_C = 800
_NCHUNK = _T // _C

_OUT_D = 164
_BBL = 256


def _sc_gather(table, idx):
    mesh = plsc.VectorSubcoreMesh(core_axis_name="c", subcore_axis_name="s")

    @functools.partial(
        pl.kernel,
        mesh=mesh,
        compiler_params=pltpu.CompilerParams(use_tc_tiling_on_sc=False),
        out_type=jax.ShapeDtypeStruct((_N, _D), jnp.float32),
        scratch_types=[
            pltpu.VMEM((_C,), jnp.int32),
            pltpu.VMEM((_C,), jnp.int32),
            pltpu.VMEM((_C, _D), jnp.float32),
            pltpu.VMEM((_C, _D), jnp.float32),
            pltpu.SemaphoreType.DMA,
            pltpu.SemaphoreType.DMA,
            pltpu.SemaphoreType.DMA,
            pltpu.SemaphoreType.DMA,
        ],
    )
    def k(table_hbm, idx_hbm, out_hbm, i0, i1, r0, r1, gs0, gs1, ws0, ws1):
        wid = lax.axis_index("s") * 2 + lax.axis_index("c")
        base0 = wid * _T
        ibufs, rbufs = [i0, i1], [r0, r1]
        gsems, wsems = [gs0, gs1], [ws0, ws1]
        gh = [None, None]
        wh = [None, None]
        pltpu.sync_copy(idx_hbm.at[pl.ds(base0, _C)], i0)
        gh[0] = pltpu.async_copy(table_hbm.at[i0], r0, gs0)
        for j in range(_NCHUNK):
            b = j % 2
            nb = (j + 1) % 2
            if j + 1 < _NCHUNK:
                pltpu.sync_copy(
                    idx_hbm.at[pl.ds(base0 + (j + 1) * _C, _C)], ibufs[nb])
                if wh[nb] is not None:
                    wh[nb].wait()
                gh[nb] = pltpu.async_copy(
                    table_hbm.at[ibufs[nb]], rbufs[nb], gsems[nb])
            gh[b].wait()
            wh[b] = pltpu.async_copy(
                rbufs[b], out_hbm.at[pl.ds(base0 + j * _C, _C)], wsems[b])
        wh[0].wait()
        wh[1].wait()

    return k(table, idx)


def _tc_assemble(idsf, bigT):
    grid = (_B // _BBL,)

    def body(fid_ref, pid_ref, prid_ref, sid_ref, flid_ref,
             xc_ref, yc_ref, ar_ref, wd_ref, ht_ref, bigT_ref, out_ref):
        BT = bigT_ref[...]
        it = lax.broadcasted_iota(jnp.int32, (32, _BBL), 0)
        one = jnp.ones((32, _BBL), jnp.float32)
        zero = jnp.zeros((32, _BBL), jnp.float32)
        for l in range(_L):
            pose = pid_ref[l:l + 1, :]
            face = fid_ref[l:l + 1, :]
            pres = prid_ref[l:l + 1, :]
            size = sid_ref[l:l + 1, :]
            flip = flid_ref[l:l + 1, :]
            F = jnp.where(
                it < 7, jnp.where(pose == it, one, zero),
                jnp.where(
                    it < 12, jnp.where(face == it - 7, one, zero),
                    jnp.where(
                        it < 14, jnp.where(pres == it - 12, one, zero),
                        jnp.where(
                            it == 14, xc_ref[l:l + 1, :],
                            jnp.where(
                                it == 15, yc_ref[l:l + 1, :],
                                jnp.where(
                                    it == 16, ar_ref[l:l + 1, :],
                                    jnp.where(
                                        it == 17, wd_ref[l:l + 1, :],
                                        jnp.where(
                                            it == 18, ht_ref[l:l + 1, :],
                                            jnp.where(
                                                it == 19, one,
                                                jnp.where(
                                                    it < 23,
                                                    jnp.where(size == it - 20, one, zero),
                                                    jnp.where(
                                                        it < 25,
                                                        jnp.where(flip == it - 23, one, zero),
                                                        zero)))))))))))
            o100 = jnp.dot(BT, F, preferred_element_type=jnp.float32)
            out_ref[l, _D:_OUT_D, :] = o100

    id_spec = pl.BlockSpec((_L, _BBL), lambda i: (0, i))
    return pl.pallas_call(
        body,
        grid=grid,
        in_specs=[
            id_spec, id_spec, id_spec, id_spec, id_spec,
            id_spec, id_spec, id_spec, id_spec, id_spec,
            pl.BlockSpec((100, 32), lambda i: (0, 0)),
        ],
        out_specs=pl.BlockSpec((_L, _OUT_D, _BBL), lambda i: (0, 0, i)),
        out_shape=jax.ShapeDtypeStruct((_L, _OUT_D, _B), jnp.float32),
        compiler_params=pltpu.CompilerParams(
            dimension_semantics=("arbitrary",),
        ),
    )(*idsf, bigT)


def _tc_clipwrite(buf, clips3):
    grid = (_B // _BBL,)

    def body(buf_ref, clips_ref, out_ref):
        del buf_ref
        for l in range(_L):
            x = clips_ref[l, :, :]
            xT = x.T
            out_ref[l, 0:_D, 0:128] = xT[0:_D, :]
            out_ref[l, 0:_D, 128:256] = xT[_D:128, :]

    return pl.pallas_call(
        body,
        grid=grid,
        in_specs=[
            pl.BlockSpec(memory_space=pl.ANY),
            pl.BlockSpec((_L, _BBL // 2, 128), lambda i: (0, i, 0)),
        ],
        out_specs=pl.BlockSpec((_L, _D, _BBL), lambda i: (0, 0, i)),
        out_shape=jax.ShapeDtypeStruct((_L, _OUT_D, _B), jnp.float32),
        input_output_aliases={0: 0},
        compiler_params=pltpu.CompilerParams(
            dimension_semantics=("arbitrary",),
        ),
    )(buf, clips3)


def kernel(clip_id, scene_face, scene_pose, scene_presence, scene_size,
           scene_flip, scene_x_center, scene_y_center, scene_area,
           scene_width, scene_height, clip_table, face_table, pose_table,
           presence_table, size_table, flip_table, pos_W, pos_b):
    idxT = (jnp.transpose(clip_id).astype(jnp.int32)
            .reshape(_L, _B // 256, 2, 128)
            .transpose(0, 1, 3, 2).reshape(_N))
    rows = _sc_gather(clip_table, idxT)
    clips3 = rows.reshape(_L, _B // 2, 128)

    Z = jnp.zeros((100, 32), jnp.float32)
    Z = Z.at[0:20, 0:7].set(pose_table.T)
    Z = Z.at[20:40, 7:12].set(face_table.T)
    Z = Z.at[40:50, 12:14].set(presence_table.T)
    Z = Z.at[50:80, 14:19].set(pos_W.T)
    Z = Z.at[50:80, 19].set(pos_b)
    Z = Z.at[80:90, 20:23].set(size_table.T)
    Z = Z.at[90:100, 23:25].set(flip_table.T)

    tr = lambda a: jnp.transpose(a)
    idsf = [tr(scene_face.astype(jnp.int32)), tr(scene_pose.astype(jnp.int32)),
            tr(scene_presence.astype(jnp.int32)), tr(scene_size.astype(jnp.int32)),
            tr(scene_flip.astype(jnp.int32)),
            tr(scene_x_center), tr(scene_y_center), tr(scene_area),
            tr(scene_width), tr(scene_height)]
    outT = _tc_assemble(idsf, Z)
    outT = _tc_clipwrite(outT, clips3)
    return jnp.transpose(outT, (2, 0, 1))


# speedup vs baseline: 1.0873x; 1.0873x over previous
import functools

import jax
import jax.numpy as jnp
from jax import lax
from jax.experimental import pallas as pl
from jax.experimental.pallas import tpu as pltpu
from jax.experimental.pallas import tpu_sc as plsc

_B, _L = 4096, 50
_N = _B * _L
_D = 64
_V = 100000
_VP = 100096
_NW = 32
_T = _N // _NW
_C = 800
_NCHUNK = _T // _C

_OUT_D = 164
_BBL = 512


def _sc_gather(table, idx):
    mesh = plsc.VectorSubcoreMesh(core_axis_name="c", subcore_axis_name="s")

    @functools.partial(
        pl.kernel,
        mesh=mesh,
        compiler_params=pltpu.CompilerParams(use_tc_tiling_on_sc=False),
        out_type=jax.ShapeDtypeStruct((_N, _D), jnp.float32),
        scratch_types=[
            pltpu.VMEM((_C,), jnp.int32),
            pltpu.VMEM((_C,), jnp.int32),
            pltpu.VMEM((_C, _D), jnp.float32),
            pltpu.VMEM((_C, _D), jnp.float32),
            pltpu.SemaphoreType.DMA,
            pltpu.SemaphoreType.DMA,
            pltpu.SemaphoreType.DMA,
            pltpu.SemaphoreType.DMA,
        ],
    )
    def k(table_hbm, idx_hbm, out_hbm, i0, i1, r0, r1, gs0, gs1, ws0, ws1):
        wid = lax.axis_index("s") * 2 + lax.axis_index("c")
        base0 = wid * _T
        ibufs, rbufs = [i0, i1], [r0, r1]
        gsems, wsems = [gs0, gs1], [ws0, ws1]
        gh = [None, None]
        wh = [None, None]
        pltpu.sync_copy(idx_hbm.at[pl.ds(base0, _C)], i0)
        gh[0] = pltpu.async_copy(table_hbm.at[i0], r0, gs0)
        for j in range(_NCHUNK):
            b = j % 2
            nb = (j + 1) % 2
            if j + 1 < _NCHUNK:
                pltpu.sync_copy(
                    idx_hbm.at[pl.ds(base0 + (j + 1) * _C, _C)], ibufs[nb])
                if wh[nb] is not None:
                    wh[nb].wait()
                gh[nb] = pltpu.async_copy(
                    table_hbm.at[ibufs[nb]], rbufs[nb], gsems[nb])
            gh[b].wait()
            wh[b] = pltpu.async_copy(
                rbufs[b], out_hbm.at[pl.ds(base0 + j * _C, _C)], wsems[b])
        wh[0].wait()
        wh[1].wait()

    return k(table, idx)


def _tc_assemble(clips3, idsf, bigT):
    grid = (_B // _BBL,)

    def body(clips_ref, fid_ref, pid_ref, prid_ref, sid_ref, flid_ref,
             xc_ref, yc_ref, ar_ref, wd_ref, ht_ref, bigT_ref, out_ref):
        BT = bigT_ref[...]
        it = lax.broadcasted_iota(jnp.int32, (32, _BBL), 0)
        one = jnp.ones((32, _BBL), jnp.float32)
        zero = jnp.zeros((32, _BBL), jnp.float32)
        for l in range(_L):
            for h in range(_BBL // 256):
                x = clips_ref[l, h * 128:(h + 1) * 128, :]
                xT = x.T
                out_ref[l, 0:_D, h * 256:h * 256 + 128] = xT[0:_D, :]
                out_ref[l, 0:_D, h * 256 + 128:h * 256 + 256] = xT[_D:128, :]

            pose = pid_ref[l:l + 1, :]
            face = fid_ref[l:l + 1, :]
            pres = prid_ref[l:l + 1, :]
            size = sid_ref[l:l + 1, :]
            flip = flid_ref[l:l + 1, :]
            F = jnp.where(
                it < 7, jnp.where(pose == it, one, zero),
                jnp.where(
                    it < 12, jnp.where(face == it - 7, one, zero),
                    jnp.where(
                        it < 14, jnp.where(pres == it - 12, one, zero),
                        jnp.where(
                            it == 14, xc_ref[l:l + 1, :],
                            jnp.where(
                                it == 15, yc_ref[l:l + 1, :],
                                jnp.where(
                                    it == 16, ar_ref[l:l + 1, :],
                                    jnp.where(
                                        it == 17, wd_ref[l:l + 1, :],
                                        jnp.where(
                                            it == 18, ht_ref[l:l + 1, :],
                                            jnp.where(
                                                it == 19, one,
                                                jnp.where(
                                                    it < 23,
                                                    jnp.where(size == it - 20, one, zero),
                                                    jnp.where(
                                                        it < 25,
                                                        jnp.where(flip == it - 23, one, zero),
                                                        zero)))))))))))
            o100 = jnp.dot(BT, F, preferred_element_type=jnp.float32)
            out_ref[l, _D:_OUT_D, :] = o100

    id_spec = pl.BlockSpec((_L, _BBL), lambda i: (0, i))
    return pl.pallas_call(
        body,
        grid=grid,
        in_specs=[
            pl.BlockSpec((_L, _BBL // 2, 128), lambda i: (0, i, 0)),
            id_spec, id_spec, id_spec, id_spec, id_spec,
            id_spec, id_spec, id_spec, id_spec, id_spec,
            pl.BlockSpec((100, 32), lambda i: (0, 0)),
        ],
        out_specs=pl.BlockSpec((_L, _OUT_D, _BBL), lambda i: (0, 0, i)),
        out_shape=jax.ShapeDtypeStruct((_L, _OUT_D, _B), jnp.float32),
        compiler_params=pltpu.CompilerParams(
            dimension_semantics=("arbitrary",),
        ),
    )(clips3, *idsf, bigT)


def kernel(clip_id, scene_face, scene_pose, scene_presence, scene_size,
           scene_flip, scene_x_center, scene_y_center, scene_area,
           scene_width, scene_height, clip_table, face_table, pose_table,
           presence_table, size_table, flip_table, pos_W, pos_b):
    idxT = (jnp.transpose(clip_id).astype(jnp.int32)
            .reshape(_L, _B // 256, 2, 128)
            .transpose(0, 1, 3, 2).reshape(_N))
    rows = _sc_gather(clip_table, idxT)
    clips3 = rows.reshape(_L, _B // 2, 128)

    Z = jnp.zeros((100, 32), jnp.float32)
    Z = Z.at[0:20, 0:7].set(pose_table.T)
    Z = Z.at[20:40, 7:12].set(face_table.T)
    Z = Z.at[40:50, 12:14].set(presence_table.T)
    Z = Z.at[50:80, 14:19].set(pos_W.T)
    Z = Z.at[50:80, 19].set(pos_b)
    Z = Z.at[80:90, 20:23].set(size_table.T)
    Z = Z.at[90:100, 23:25].set(flip_table.T)

    tr = lambda a: jnp.transpose(a)
    idsf = [tr(scene_face.astype(jnp.int32)), tr(scene_pose.astype(jnp.int32)),
            tr(scene_presence.astype(jnp.int32)), tr(scene_size.astype(jnp.int32)),
            tr(scene_flip.astype(jnp.int32)),
            tr(scene_x_center), tr(scene_y_center), tr(scene_area),
            tr(scene_width), tr(scene_height)]
    outT = _tc_assemble(clips3, idsf, Z)
    return jnp.transpose(outT, (2, 0, 1))


# speedup vs baseline: 1.0888x; 1.0013x over previous
import functools

import jax
import jax.numpy as jnp
from jax import lax
from jax.experimental import pallas as pl
from jax.experimental.pallas import tpu as pltpu
from jax.experimental.pallas import tpu_sc as plsc

_B, _L = 4096, 50
_N = _B * _L
_D = 64
_V = 100000
_NW = 32
_T = _N // _NW
_C = 800
_NCHUNK = _T // _C

_OUT_D = 164
_BBL = 512


def _sc_gather(table, idx):
    mesh = plsc.VectorSubcoreMesh(core_axis_name="c", subcore_axis_name="s")

    @functools.partial(
        pl.kernel,
        mesh=mesh,
        compiler_params=pltpu.CompilerParams(use_tc_tiling_on_sc=False),
        out_type=jax.ShapeDtypeStruct((_N, _D), jnp.float32),
        scratch_types=[
            pltpu.VMEM((_C,), jnp.int32),
            pltpu.VMEM((_C,), jnp.int32),
            pltpu.VMEM((_C, _D), jnp.float32),
            pltpu.VMEM((_C, _D), jnp.float32),
            pltpu.SemaphoreType.DMA,
            pltpu.SemaphoreType.DMA,
            pltpu.SemaphoreType.DMA,
            pltpu.SemaphoreType.DMA,
        ],
    )
    def k(table_hbm, idx_hbm, out_hbm, i0, i1, r0, r1, gs0, gs1, ws0, ws1):
        wid = lax.axis_index("s") * 2 + lax.axis_index("c")
        base0 = wid * _T
        ibufs, rbufs = [i0, i1], [r0, r1]
        gsems, wsems = [gs0, gs1], [ws0, ws1]
        gh = [None, None]
        wh = [None, None]
        pltpu.sync_copy(idx_hbm.at[pl.ds(base0, _C)], i0)
        gh[0] = pltpu.async_copy(table_hbm.at[i0], r0, gs0)
        for j in range(_NCHUNK):
            b = j % 2
            nb = (j + 1) % 2
            if j + 1 < _NCHUNK:
                pltpu.sync_copy(
                    idx_hbm.at[pl.ds(base0 + (j + 1) * _C, _C)], ibufs[nb])
                if wh[nb] is not None:
                    wh[nb].wait()
                gh[nb] = pltpu.async_copy(
                    table_hbm.at[ibufs[nb]], rbufs[nb], gsems[nb])
            gh[b].wait()
            wh[b] = pltpu.async_copy(
                rbufs[b], out_hbm.at[pl.ds(base0 + j * _C, _C)], wsems[b])
        wh[0].wait()
        wh[1].wait()

    return k(table, idx)


def _tc_assemble(clips3, idsf, bigT):
    grid = (_B // _BBL,)

    def body(clips_ref, fid_ref, pid_ref, prid_ref, sid_ref, flid_ref,
             xc_ref, yc_ref, ar_ref, wd_ref, ht_ref, bigT_ref, out_ref):
        BT = bigT_ref[...]
        it = lax.broadcasted_iota(jnp.int32, (32, _BBL), 0)
        one = jnp.ones((32, _BBL), jnp.float32)
        zero = jnp.zeros((32, _BBL), jnp.float32)
        for l in range(_L):
            for h in range(_BBL // 256):
                x = clips_ref[l, h * 128:(h + 1) * 128, :]
                xT = x.T
                out_ref[l, 0:_D, h * 256:h * 256 + 128] = xT[0:_D, :]
                out_ref[l, 0:_D, h * 256 + 128:h * 256 + 256] = xT[_D:128, :]

            pose = pid_ref[l:l + 1, :]
            face = fid_ref[l:l + 1, :]
            pres = prid_ref[l:l + 1, :]
            size = sid_ref[l:l + 1, :]
            flip = flid_ref[l:l + 1, :]
            F = jnp.where(
                it < 7, jnp.where(pose == it, one, zero),
                jnp.where(
                    it < 12, jnp.where(face == it - 7, one, zero),
                    jnp.where(
                        it < 14, jnp.where(pres == it - 12, one, zero),
                        jnp.where(
                            it == 14, xc_ref[l:l + 1, :],
                            jnp.where(
                                it == 15, yc_ref[l:l + 1, :],
                                jnp.where(
                                    it == 16, ar_ref[l:l + 1, :],
                                    jnp.where(
                                        it == 17, wd_ref[l:l + 1, :],
                                        jnp.where(
                                            it == 18, ht_ref[l:l + 1, :],
                                            jnp.where(
                                                it == 19, one,
                                                jnp.where(
                                                    it < 23,
                                                    jnp.where(size == it - 20, one, zero),
                                                    jnp.where(
                                                        it < 25,
                                                        jnp.where(flip == it - 23, one, zero),
                                                        zero)))))))))))
            o100 = jnp.dot(BT, F, preferred_element_type=jnp.float32)
            out_ref[l, _D:_OUT_D, :] = o100

    id_spec = pl.BlockSpec((_L, _BBL), lambda i: (0, i))
    return pl.pallas_call(
        body,
        grid=grid,
        in_specs=[
            pl.BlockSpec((_L, _BBL // 2, 128), lambda i: (0, i, 0)),
            id_spec, id_spec, id_spec, id_spec, id_spec,
            id_spec, id_spec, id_spec, id_spec, id_spec,
            pl.BlockSpec((100, 32), lambda i: (0, 0)),
        ],
        out_specs=pl.BlockSpec((_L, _OUT_D, _BBL), lambda i: (0, 0, i)),
        out_shape=jax.ShapeDtypeStruct((_L, _OUT_D, _B), jnp.float32),
        compiler_params=pltpu.CompilerParams(
            dimension_semantics=("arbitrary",),
        ),
    )(clips3, *idsf, bigT)


def kernel(clip_id, scene_face, scene_pose, scene_presence, scene_size,
           scene_flip, scene_x_center, scene_y_center, scene_area,
           scene_width, scene_height, clip_table, face_table, pose_table,
           presence_table, size_table, flip_table, pos_W, pos_b):
    idxT = (jnp.transpose(clip_id).astype(jnp.int32)
            .reshape(_L, _B // 256, 2, 128)
            .transpose(0, 1, 3, 2).reshape(_N))
    rows = _sc_gather(clip_table, idxT)
    clips3 = rows.reshape(_L, _B // 2, 128)

    Z = jnp.zeros((100, 32), jnp.float32)
    Z = Z.at[0:20, 0:7].set(pose_table.T)
    Z = Z.at[20:40, 7:12].set(face_table.T)
    Z = Z.at[40:50, 12:14].set(presence_table.T)
    Z = Z.at[50:80, 14:19].set(pos_W.T)
    Z = Z.at[50:80, 19].set(pos_b)
    Z = Z.at[80:90, 20:23].set(size_table.T)
    Z = Z.at[90:100, 23:25].set(flip_table.T)

    tr = lambda a: jnp.transpose(a)
    idsf = [tr(scene_face.astype(jnp.int32)), tr(scene_pose.astype(jnp.int32)),
            tr(scene_presence.astype(jnp.int32)), tr(scene_size.astype(jnp.int32)),
            tr(scene_flip.astype(jnp.int32)),
            tr(scene_x_center), tr(scene_y_center), tr(scene_area),
            tr(scene_width), tr(scene_height)]
    outT = _tc_assemble(clips3, idsf, Z)
    return jnp.transpose(outT, (2, 0, 1))
